# baseline (speedup 1.0000x reference)
import functools
import math

import jax
import jax.numpy as jnp
from jax import lax
from jax.experimental import pallas as pl
from jax.experimental.pallas import tpu as pltpu
from jax.experimental.pallas import tpu_sc as plsc

SIZE = 32000
N_TOK = 4096
SMOOTHING = 0.1
CONFIDENCE = 1.0 - SMOOTHING
U = SMOOTHING / (SIZE - 2)
K_CONST = (SIZE - 2) * U * math.log(U) + CONFIDENCE * math.log(CONFIDENCE)

LANES = 16
NW = 32

NT_SC = 1856
NT_TC = N_TOK - NT_SC
RPW = NT_SC // NW

BN = 640
GRID_J = SIZE // BN

INNER_UNROLL = 16
INNER_ITERS = SIZE // LANES // INNER_UNROLL


def _onehot(l):
    return jnp.where(lax.iota(jnp.int32, LANES) == l,
                     jnp.float32(1.0), jnp.float32(0.0))


def _tc_body(t_ref, x_ref, out_ref):
    j = pl.program_id(0)

    @pl.when(j == 0)
    def _init():
        out_ref[...] = jnp.zeros_like(out_ref)

    tile = x_ref[...]
    tcol = t_ref[...]
    mask = (tcol != 0).astype(jnp.float32)
    rowsum = jnp.sum(tile, axis=1, keepdims=True)
    val = jnp.float32(-U) * jnp.sum(rowsum * mask)
    cols = lax.broadcasted_iota(jnp.int32, (NT_TC, BN), 1) + j * BN
    hit = jnp.where((cols == tcol) & (tcol != 0), jnp.float32(1.0),
                    jnp.float32(0.0))
    val = val + jnp.float32(-(CONFIDENCE - U)) * jnp.sum(tile * hit)
    extra = (jnp.float32(U) * jnp.sum(tile[:, 0:1] * mask)
             + jnp.float32(K_CONST) * jnp.sum(mask))
    val = val + jnp.where(j == 0, extra, jnp.float32(0.0))
    out_ref[...] += val


def _tc_reduce(x, t2d):
    return pl.pallas_call(
        _tc_body,
        grid=(GRID_J,),
        in_specs=[
            pl.BlockSpec((NT_TC, 1), lambda j: (0, 0)),
            pl.BlockSpec((NT_TC, BN), lambda j: (0, j)),
        ],
        out_specs=pl.BlockSpec((1, 1), lambda j: (0, 0)),
        out_shape=jax.ShapeDtypeStruct((1, 1), jnp.float32),
    )(t2d, x)


def _row_scan(buf, tvs):
    zero = jnp.zeros((LANES,), jnp.float32)
    diff = tvs - lax.iota(jnp.int32, LANES)

    def body(k, carry):
        accs = list(carry[:4])
        gsel = list(carry[4:])
        base = k * (INNER_UNROLL * LANES)
        for u in range(INNER_UNROLL):
            col0 = base + u * LANES
            c = buf[pl.ds(col0, LANES)]
            accs[u % 4] = accs[u % 4] + c
            gsel[u % 4] = jnp.where(diff == col0, gsel[u % 4] + c,
                                    gsel[u % 4])
        return tuple(accs) + tuple(gsel)

    res = lax.fori_loop(
        0, INNER_ITERS, body, (zero,) * 8)
    return ((res[0] + res[1]) + (res[2] + res[3]),
            (res[4] + res[5]) + (res[6] + res[7]))


def _sc_body(x_hbm, tsp_hbm, out_hbm,
             tgtd_v, buf0, buf1, gv_v, acc_v, semg, sem0, sem1):
    wid = lax.axis_index("s") * 2 + lax.axis_index("c")
    dbase = NT_TC + wid * RPW
    pltpu.sync_copy(tsp_hbm.at[pl.ds(wid * RPW * LANES, RPW * LANES)], tgtd_v)
    bufs = (buf0, buf1)
    sems = (sem0, sem1)
    copies = [None, None]
    H = SIZE // 2

    def _start(r, b):
        c0 = pltpu.async_copy(
            x_hbm.at[dbase + r, pl.ds(0, H)], bufs[b].at[pl.ds(0, H)],
            sems[b])
        c1 = pltpu.async_copy(
            x_hbm.at[dbase + r, pl.ds(H, H)], bufs[b].at[pl.ds(H, H)],
            sems[b])
        return (c0, c1)

    copies[0] = _start(0, 0)
    total = jnp.zeros((LANES,), jnp.float32)
    for r in range(RPW):
        b = r & 1
        if r + 1 < RPW:
            nb = (r + 1) & 1
            copies[nb] = _start(r + 1, nb)
        copies[b][0].wait()
        copies[b][1].wait()
        buf = bufs[b]
        tvs = tgtd_v[pl.ds(r * LANES, LANES)]
        acc16, gsel = _row_scan(buf, tvs)
        head = buf[pl.ds(0, LANES)]
        e0 = _onehot(0)
        term = (jnp.float32(-U) * acc16
                + jnp.float32(-(CONFIDENCE - U)) * gsel
                + e0 * (jnp.float32(U) * head
                        + jnp.float32(K_CONST)))
        total = total + jnp.where(tvs != 0, term, jnp.float32(0.0))
    acc_v[...] = total
    pltpu.sync_copy(acc_v, out_hbm.at[wid])


@functools.lru_cache(maxsize=1)
def _make_sc_kernel():
    return pl.kernel(
        _sc_body,
        out_type=jax.ShapeDtypeStruct((NW, LANES), jnp.float32),
        scratch_types=[
            pltpu.VMEM((RPW * LANES,), jnp.int32),
            pltpu.VMEM((SIZE,), jnp.float32),
            pltpu.VMEM((SIZE,), jnp.float32),
            pltpu.VMEM((LANES,), jnp.float32),
            pltpu.VMEM((LANES,), jnp.float32),
            pltpu.SemaphoreType.DMA,
            pltpu.SemaphoreType.DMA,
            pltpu.SemaphoreType.DMA,
        ],
        mesh=plsc.VectorSubcoreMesh(core_axis_name="c", subcore_axis_name="s"),
    )


@jax.jit
def kernel(x, target):
    tsplat = jnp.broadcast_to(target[NT_TC:, None], (NT_SC, LANES)).reshape(-1)
    sc_part = _make_sc_kernel()(x, tsplat)
    tc_part = _tc_reduce(x, target[:NT_TC].reshape(NT_TC, 1))
    return tc_part[0, 0] + jnp.sum(sc_part)


# speedup vs baseline: 1.0258x; 1.0258x over previous
import functools
import math

import jax
import jax.numpy as jnp
from jax import lax
from jax.experimental import pallas as pl
from jax.experimental.pallas import tpu as pltpu
from jax.experimental.pallas import tpu_sc as plsc

SIZE = 32000
N_TOK = 4096
SMOOTHING = 0.1
CONFIDENCE = 1.0 - SMOOTHING
U = SMOOTHING / (SIZE - 2)
K_CONST = (SIZE - 2) * U * math.log(U) + CONFIDENCE * math.log(CONFIDENCE)

LANES = 16
NW = 32

NT_SC = 1856
NT_TC = N_TOK - NT_SC
RPW = NT_SC // NW

BN = 1280
GRID_J = SIZE // BN

INNER_UNROLL = 16
INNER_ITERS = SIZE // LANES // INNER_UNROLL


def _onehot(l):
    return jnp.where(lax.iota(jnp.int32, LANES) == l,
                     jnp.float32(1.0), jnp.float32(0.0))


def _tc_body(t_ref, x_ref, out_ref):
    j = pl.program_id(0)

    @pl.when(j == 0)
    def _init():
        out_ref[...] = jnp.zeros_like(out_ref)

    tile = x_ref[...]
    tcol = t_ref[...]
    mask = (tcol != 0).astype(jnp.float32)
    rowsum = jnp.sum(tile, axis=1, keepdims=True)
    val = jnp.float32(-U) * jnp.sum(rowsum * mask)
    cols = lax.broadcasted_iota(jnp.int32, (NT_TC, BN), 1) + j * BN
    hit = jnp.where((cols == tcol) & (tcol != 0), jnp.float32(1.0),
                    jnp.float32(0.0))
    val = val + jnp.float32(-(CONFIDENCE - U)) * jnp.sum(tile * hit)
    extra = (jnp.float32(U) * jnp.sum(tile[:, 0:1] * mask)
             + jnp.float32(K_CONST) * jnp.sum(mask))
    val = val + jnp.where(j == 0, extra, jnp.float32(0.0))
    out_ref[...] += val


def _tc_reduce(x, t2d):
    return pl.pallas_call(
        _tc_body,
        grid=(GRID_J,),
        in_specs=[
            pl.BlockSpec((NT_TC, 1), lambda j: (0, 0)),
            pl.BlockSpec((NT_TC, BN), lambda j: (0, j)),
        ],
        out_specs=pl.BlockSpec((1, 1), lambda j: (0, 0)),
        out_shape=jax.ShapeDtypeStruct((1, 1), jnp.float32),
    )(t2d, x)


def _row_scan(buf, tvs):
    zero = jnp.zeros((LANES,), jnp.float32)
    diff = tvs - lax.iota(jnp.int32, LANES)

    def body(k, carry):
        accs = list(carry[:4])
        gsel = list(carry[4:])
        base = k * (INNER_UNROLL * LANES)
        for u in range(INNER_UNROLL):
            col0 = base + u * LANES
            c = buf[pl.ds(col0, LANES)]
            accs[u % 4] = accs[u % 4] + c
            gsel[u % 4] = jnp.where(diff == col0, gsel[u % 4] + c,
                                    gsel[u % 4])
        return tuple(accs) + tuple(gsel)

    res = lax.fori_loop(
        0, INNER_ITERS, body, (zero,) * 8)
    return ((res[0] + res[1]) + (res[2] + res[3]),
            (res[4] + res[5]) + (res[6] + res[7]))


def _sc_body(x_hbm, tsp_hbm, out_hbm,
             tgtd_v, buf0, buf1, acc_v, sem0, sem1):
    wid = lax.axis_index("s") * 2 + lax.axis_index("c")
    dbase = NT_TC + wid * RPW
    pltpu.sync_copy(tsp_hbm.at[pl.ds(wid * RPW * LANES, RPW * LANES)], tgtd_v)
    bufs = (buf0, buf1)
    sems = (sem0, sem1)
    copies = [None, None]
    H = SIZE // 2

    def _start(r, b):
        c0 = pltpu.async_copy(
            x_hbm.at[dbase + r, pl.ds(0, H)], bufs[b].at[pl.ds(0, H)],
            sems[b])
        c1 = pltpu.async_copy(
            x_hbm.at[dbase + r, pl.ds(H, H)], bufs[b].at[pl.ds(H, H)],
            sems[b])
        return (c0, c1)

    copies[0] = _start(0, 0)
    total = jnp.zeros((LANES,), jnp.float32)
    for r in range(RPW):
        b = r & 1
        if r + 1 < RPW:
            nb = (r + 1) & 1
            copies[nb] = _start(r + 1, nb)
        copies[b][0].wait()
        copies[b][1].wait()
        buf = bufs[b]
        tvs = tgtd_v[pl.ds(r * LANES, LANES)]
        acc16, gsel = _row_scan(buf, tvs)
        head = buf[pl.ds(0, LANES)]
        e0 = _onehot(0)
        term = (jnp.float32(-U) * acc16
                + jnp.float32(-(CONFIDENCE - U)) * gsel
                + e0 * (jnp.float32(U) * head
                        + jnp.float32(K_CONST)))
        total = total + jnp.where(tvs != 0, term, jnp.float32(0.0))
    acc_v[...] = total
    pltpu.sync_copy(acc_v, out_hbm.at[wid])


@functools.lru_cache(maxsize=1)
def _make_sc_kernel():
    return pl.kernel(
        _sc_body,
        out_type=jax.ShapeDtypeStruct((NW, LANES), jnp.float32),
        scratch_types=[
            pltpu.VMEM((RPW * LANES,), jnp.int32),
            pltpu.VMEM((SIZE,), jnp.float32),
            pltpu.VMEM((SIZE,), jnp.float32),
            pltpu.VMEM((LANES,), jnp.float32),
            pltpu.SemaphoreType.DMA,
            pltpu.SemaphoreType.DMA,
        ],
        mesh=plsc.VectorSubcoreMesh(core_axis_name="c", subcore_axis_name="s"),
    )


@jax.jit
def kernel(x, target):
    tsplat = jnp.broadcast_to(target[NT_TC:, None], (NT_SC, LANES)).reshape(-1)
    sc_part = _make_sc_kernel()(x, tsplat)
    tc_part = _tc_reduce(x, target[:NT_TC].reshape(NT_TC, 1))
    return tc_part[0, 0] + jnp.sum(sc_part)
